# baseline (speedup 1.0000x reference)
import functools

import jax
import jax.numpy as jnp
from jax import lax
from jax.experimental import pallas as pl
from jax.experimental.pallas import tpu as pltpu
from jax.experimental.pallas import tpu_sc as plsc

_B, _N, _M = 8, 2048, 2048
_NW = 32
_QW = 4
_RW = _N // _QW
_RB = 8
_L = 16
_BIG = 3.0e38


def _lane_min(v):
    vals = [v[l] for l in range(_L)]
    while len(vals) > 1:
        vals = [
            jnp.minimum(vals[2 * i], vals[2 * i + 1])
            for i in range(len(vals) // 2)
        ]
    return vals[0]


def _sc_body(p_hbm, g_hbm, d1_hbm, cm_hbm, g_v, p_v, cm_v, r_v):
    c = lax.axis_index("c")
    s = lax.axis_index("s")
    wid = c * 16 + s
    b = wid // _QW
    q = wid % _QW
    pltpu.sync_copy(g_hbm.at[b], g_v)
    pltpu.sync_copy(p_hbm.at[b, :, pl.ds(q * _RW, _RW)], p_v)

    def init_cm(i, carry):
        cm_v[pl.ds(i * _L, _L)] = jnp.full((_L,), _BIG, jnp.float32)
        return carry

    lax.fori_loop(0, _M // _L, init_cm, 0)

    lane = lax.iota(jnp.int32, _L)

    def row_block(rb, carry):
        base = rb * _L
        px = p_v[0, pl.ds(base, _L)]
        py = p_v[1, pl.ds(base, _L)]
        pz = p_v[2, pl.ds(base, _L)]
        rowmins = jnp.full((_L,), 0.0, jnp.float32)
        for sub in range(_L // _RB):
            bx = []
            by = []
            bz = []
            for r in range(_RB):
                j = sub * _RB + r
                bx.append(jnp.full((_L,), px[j]))
                by.append(jnp.full((_L,), py[j]))
                bz.append(jnp.full((_L,), pz[j]))

            def chunk(ci, accs):
                o = ci * _L
                gx = g_v[0, pl.ds(o, _L)]
                gy = g_v[1, pl.ds(o, _L)]
                gz = g_v[2, pl.ds(o, _L)]
                cm = cm_v[pl.ds(o, _L)]
                out = []
                for r in range(_RB):
                    dx = gx - bx[r]
                    dy = gy - by[r]
                    dz = gz - bz[r]
                    dd = dx * dx + dy * dy + dz * dz
                    cm = jnp.minimum(cm, dd)
                    out.append(jnp.minimum(accs[r], dd))
                cm_v[pl.ds(o, _L)] = cm
                return tuple(out)

            accs = lax.fori_loop(
                0, _M // _L, chunk,
                tuple(jnp.full((_L,), _BIG, jnp.float32) for _ in range(_RB)),
            )
            for r in range(_RB):
                j = sub * _RB + r
                m = jnp.maximum(_lane_min(accs[r]), 0.0)
                rowmins = jnp.where(lane == j, m, rowmins)
        r_v[pl.ds(base, _L)] = rowmins
        return carry

    lax.fori_loop(0, _RW // _L, row_block, 0)

    pltpu.sync_copy(r_v, d1_hbm.at[b, pl.ds(q * _RW, _RW)])
    pltpu.sync_copy(cm_v, cm_hbm.at[b, q])


_sc_chamfer = functools.partial(
    pl.kernel,
    mesh=plsc.VectorSubcoreMesh(core_axis_name="c", subcore_axis_name="s"),
    out_type=[
        jax.ShapeDtypeStruct((_B, _N), jnp.float32),
        jax.ShapeDtypeStruct((_B, _QW, _M), jnp.float32),
    ],
    scratch_types=[
        pltpu.VMEM((3, _M), jnp.float32),
        pltpu.VMEM((3, _RW), jnp.float32),
        pltpu.VMEM((_M,), jnp.float32),
        pltpu.VMEM((_RW,), jnp.float32),
    ],
)(_sc_body)


def kernel(pred_points, gt_points):
    p_t = jnp.transpose(pred_points, (0, 2, 1))
    g_t = jnp.transpose(gt_points, (0, 2, 1))
    d1, cm = _sc_chamfer(p_t, g_t)
    d2 = jnp.maximum(jnp.min(cm, axis=1), 0.0)
    return jnp.mean(d1) + jnp.mean(d2)


# speedup vs baseline: 2.6453x; 2.6453x over previous
import functools

import jax
import jax.numpy as jnp
from jax import lax
from jax.experimental import pallas as pl
from jax.experimental.pallas import tpu as pltpu
from jax.experimental.pallas import tpu_sc as plsc

_B, _N, _M = 8, 2048, 2048
_L = 16
_RB = 8
_BIG = 3.0e38

_NB_SC = 2
_NB_TC = _B - _NB_SC
_WPB = 32 // _NB_SC
_RW = _N // _WPB


def _lane_min(v):
    vals = [v[l] for l in range(_L)]
    while len(vals) > 1:
        vals = [
            jnp.minimum(vals[2 * i], vals[2 * i + 1])
            for i in range(len(vals) // 2)
        ]
    return vals[0]


def _sc_body(p_hbm, g_hbm, d1_hbm, cm_hbm, g_v, p_v, cm_v, r_v):
    c = lax.axis_index("c")
    s = lax.axis_index("s")
    wid = c * 16 + s
    b = wid // _WPB
    q = wid % _WPB
    pltpu.sync_copy(g_hbm.at[b], g_v)
    pltpu.sync_copy(p_hbm.at[b, :, pl.ds(q * _RW, _RW)], p_v)

    def init_cm(i, carry):
        cm_v[pl.ds(i * _L, _L)] = jnp.full((_L,), _BIG, jnp.float32)
        return carry

    lax.fori_loop(0, _M // _L, init_cm, 0)

    lane = lax.iota(jnp.int32, _L)

    def row_block(rb, carry):
        base = rb * _L
        px = p_v[0, pl.ds(base, _L)]
        py = p_v[1, pl.ds(base, _L)]
        pz = p_v[2, pl.ds(base, _L)]
        rowmins = jnp.full((_L,), 0.0, jnp.float32)
        for sub in range(_L // _RB):
            bx = []
            by = []
            bz = []
            for r in range(_RB):
                j = sub * _RB + r
                bx.append(jnp.full((_L,), px[j]))
                by.append(jnp.full((_L,), py[j]))
                bz.append(jnp.full((_L,), pz[j]))

            def chunk(ci, accs):
                o = ci * _L
                gx = g_v[0, pl.ds(o, _L)]
                gy = g_v[1, pl.ds(o, _L)]
                gz = g_v[2, pl.ds(o, _L)]
                dds = []
                for r in range(_RB):
                    dx = gx - bx[r]
                    dy = gy - by[r]
                    dz = gz - bz[r]
                    dds.append(dx * dx + dy * dy + dz * dz)
                t = list(dds)
                while len(t) > 1:
                    t = [
                        jnp.minimum(t[2 * i], t[2 * i + 1])
                        for i in range(len(t) // 2)
                    ]
                cm_v[pl.ds(o, _L)] = jnp.minimum(cm_v[pl.ds(o, _L)], t[0])
                return tuple(
                    jnp.minimum(accs[r], dds[r]) for r in range(_RB)
                )

            accs = lax.fori_loop(
                0, _M // _L, chunk,
                tuple(jnp.full((_L,), _BIG, jnp.float32) for _ in range(_RB)),
            )
            for r in range(_RB):
                j = sub * _RB + r
                m = jnp.maximum(_lane_min(accs[r]), 0.0)
                rowmins = jnp.where(lane == j, m, rowmins)
        r_v[pl.ds(base, _L)] = rowmins
        return carry

    lax.fori_loop(0, _RW // _L, row_block, 0)

    pltpu.sync_copy(r_v, d1_hbm.at[b, pl.ds(q * _RW, _RW)])
    pltpu.sync_copy(cm_v, cm_hbm.at[b, q])


_sc_chamfer = functools.partial(
    pl.kernel,
    mesh=plsc.VectorSubcoreMesh(core_axis_name="c", subcore_axis_name="s"),
    out_type=[
        jax.ShapeDtypeStruct((_NB_SC, _N), jnp.float32),
        jax.ShapeDtypeStruct((_NB_SC, _WPB, _M), jnp.float32),
    ],
    scratch_types=[
        pltpu.VMEM((3, _M), jnp.float32),
        pltpu.VMEM((3, _RW), jnp.float32),
        pltpu.VMEM((_M,), jnp.float32),
        pltpu.VMEM((_RW,), jnp.float32),
    ],
)(_sc_body)


_RT = 512
_NI = _N // _RT


def _tc_body(p_ref, g_ref, d1_ref, d2_ref):
    i = pl.program_id(1)
    p = p_ref[0]
    g = g_ref[0]
    p2 = jnp.sum(p * p, axis=0, keepdims=True)
    g2 = jnp.sum(g * g, axis=0, keepdims=True)
    p2_hi = p2.astype(jnp.bfloat16).astype(jnp.float32)
    p2_lo = p2 - p2_hi
    g2_hi = g2.astype(jnp.bfloat16).astype(jnp.float32)
    g2_lo = g2 - g2_hi
    ones_p = jnp.ones_like(p2)
    ones_g = jnp.ones_like(g2)
    lhs = jnp.concatenate([p, p2_hi, p2_lo, ones_p, ones_p], axis=0)
    rhs = jnp.concatenate(
        [-2.0 * g, ones_g, ones_g, g2_hi, g2_lo], axis=0
    )
    d = jax.lax.dot_general(
        lhs, rhs, (((0,), (0,)), ((), ())), preferred_element_type=jnp.float32
    )
    d1_ref[0, 0] = jnp.maximum(jnp.min(d, axis=1), 0.0)
    colmin = jnp.min(d, axis=0)

    @pl.when(i == 0)
    def _():
        d2_ref[0, 0] = colmin

    @pl.when(i > 0)
    def _():
        d2_ref[0, 0] = jnp.minimum(d2_ref[0, 0], colmin)


def _tc_chamfer(p_t, g_t):
    return pl.pallas_call(
        _tc_body,
        grid=(_NB_TC, _NI),
        in_specs=[
            pl.BlockSpec((1, 3, _RT), lambda b, i: (b, 0, i)),
            pl.BlockSpec((1, 3, _M), lambda b, i: (b, 0, 0)),
        ],
        out_specs=[
            pl.BlockSpec((1, 1, _RT), lambda b, i: (b, 0, i)),
            pl.BlockSpec((1, 1, _M), lambda b, i: (b, 0, 0)),
        ],
        out_shape=[
            jax.ShapeDtypeStruct((_NB_TC, 1, _N), jnp.float32),
            jax.ShapeDtypeStruct((_NB_TC, 1, _M), jnp.float32),
        ],
    )(p_t, g_t)


def kernel(pred_points, gt_points):
    p_t = jnp.transpose(pred_points, (0, 2, 1))
    g_t = jnp.transpose(gt_points, (0, 2, 1))
    d1_sc, cm_sc = _sc_chamfer(p_t[:_NB_SC], g_t[:_NB_SC])
    d1_tc, d2_tc = _tc_chamfer(p_t[_NB_SC:], g_t[_NB_SC:])
    d2_sc = jnp.maximum(jnp.min(cm_sc, axis=1), 0.0)
    d1 = jnp.concatenate([d1_sc, d1_tc[:, 0, :]], axis=0)
    d2 = jnp.concatenate([d2_sc, jnp.maximum(d2_tc[:, 0, :], 0.0)], axis=0)
    return jnp.mean(d1) + jnp.mean(d2)


# speedup vs baseline: 3.4812x; 1.3160x over previous
import functools

import jax
import jax.numpy as jnp
from jax import lax
from jax.experimental import pallas as pl
from jax.experimental.pallas import tpu as pltpu
from jax.experimental.pallas import tpu_sc as plsc

_B, _N, _M = 8, 2048, 2048
_L = 16
_RB = 8
_BIG = 3.0e38

_NB_SC = 1
_NB_TC = _B - _NB_SC
_WPB = 32 // _NB_SC
_RW = _N // _WPB


def _lane_min(v):
    vals = [v[l] for l in range(_L)]
    while len(vals) > 1:
        vals = [
            jnp.minimum(vals[2 * i], vals[2 * i + 1])
            for i in range(len(vals) // 2)
        ]
    return vals[0]


def _sc_body(p_hbm, g_hbm, d1_hbm, cm_hbm, g_v, p_v, cm_v, r_v, g2_v):
    c = lax.axis_index("c")
    s = lax.axis_index("s")
    wid = c * 16 + s
    b = wid // _WPB
    q = wid % _WPB
    pltpu.sync_copy(g_hbm.at[b], g_v)
    pltpu.sync_copy(p_hbm.at[b], p_v)
    row0 = q * _RW

    def init_cm(i, carry):
        o = i * _L
        cm_v[pl.ds(o, _L)] = jnp.full((_L,), _BIG, jnp.float32)
        gx = g_v[0, pl.ds(o, _L)]
        gy = g_v[1, pl.ds(o, _L)]
        gz = g_v[2, pl.ds(o, _L)]
        g2_v[pl.ds(o, _L)] = gx * gx + (gy * gy + gz * gz)
        return carry

    lax.fori_loop(0, _M // _L, init_cm, 0)

    lane = lax.iota(jnp.int32, _L)

    def row_block(rb, carry):
        base = row0 + rb * _L
        px = p_v[0, pl.ds(base, _L)]
        py = p_v[1, pl.ds(base, _L)]
        pz = p_v[2, pl.ds(base, _L)]
        p2 = px * px + (py * py + pz * pz)
        rowmins = jnp.full((_L,), 0.0, jnp.float32)
        for sub in range(_L // _RB):
            bx = []
            by = []
            bz = []
            bp = []
            for r in range(_RB):
                j = sub * _RB + r
                bx.append(jnp.full((_L,), -2.0 * px[j]))
                by.append(jnp.full((_L,), -2.0 * py[j]))
                bz.append(jnp.full((_L,), -2.0 * pz[j]))
                bp.append(jnp.full((_L,), p2[j]))

            def chunk(ci, accs):
                o = ci * _L
                gx = g_v[0, pl.ds(o, _L)]
                gy = g_v[1, pl.ds(o, _L)]
                gz = g_v[2, pl.ds(o, _L)]
                g2c = g2_v[pl.ds(o, _L)]
                dds = []
                for r in range(_RB):
                    dds.append(
                        bx[r] * gx + (by[r] * gy + (bz[r] * gz + (g2c + bp[r])))
                    )
                t = list(dds)
                while len(t) > 1:
                    t = [
                        jnp.minimum(t[2 * i], t[2 * i + 1])
                        for i in range(len(t) // 2)
                    ]
                cm_v[pl.ds(o, _L)] = jnp.minimum(cm_v[pl.ds(o, _L)], t[0])
                return tuple(
                    jnp.minimum(accs[r], dds[r]) for r in range(_RB)
                )

            accs = lax.fori_loop(
                0, _M // _L, chunk,
                tuple(jnp.full((_L,), _BIG, jnp.float32) for _ in range(_RB)),
            )
            for r in range(_RB):
                j = sub * _RB + r
                m = jnp.maximum(_lane_min(accs[r]), 0.0)
                rowmins = jnp.where(lane == j, m, rowmins)
        r_v[pl.ds(rb * _L, _L)] = rowmins
        return carry

    lax.fori_loop(0, _RW // _L, row_block, 0)

    pltpu.sync_copy(r_v, d1_hbm.at[b, pl.ds(q * _RW, _RW)])
    pltpu.sync_copy(cm_v, cm_hbm.at[b, q])


_sc_chamfer = functools.partial(
    pl.kernel,
    mesh=plsc.VectorSubcoreMesh(core_axis_name="c", subcore_axis_name="s"),
    out_type=[
        jax.ShapeDtypeStruct((_NB_SC, _N), jnp.float32),
        jax.ShapeDtypeStruct((_NB_SC, _WPB, _M), jnp.float32),
    ],
    scratch_types=[
        pltpu.VMEM((3, _M), jnp.float32),
        pltpu.VMEM((3, _N), jnp.float32),
        pltpu.VMEM((_M,), jnp.float32),
        pltpu.VMEM((_RW,), jnp.float32),
        pltpu.VMEM((_M,), jnp.float32),
    ],
)(_sc_body)


_RT = 512
_NI = _N // _RT


def _tc_body(p_ref, g_ref, d1_ref, d2_ref):
    i = pl.program_id(1)
    p = p_ref[0]
    g = g_ref[0]
    p2 = jnp.sum(p * p, axis=0, keepdims=True)
    g2 = jnp.sum(g * g, axis=0, keepdims=True)
    p2_hi = p2.astype(jnp.bfloat16).astype(jnp.float32)
    p2_lo = p2 - p2_hi
    g2_hi = g2.astype(jnp.bfloat16).astype(jnp.float32)
    g2_lo = g2 - g2_hi
    ones_p = jnp.ones_like(p2)
    ones_g = jnp.ones_like(g2)
    lhs = jnp.concatenate([p, p2_hi, p2_lo, ones_p, ones_p], axis=0)
    rhs = jnp.concatenate(
        [-2.0 * g, ones_g, ones_g, g2_hi, g2_lo], axis=0
    )
    d = jax.lax.dot_general(
        lhs, rhs, (((0,), (0,)), ((), ())), preferred_element_type=jnp.float32
    )
    part = d[:, 0:128]
    for k in range(1, _M // 128):
        part = jnp.minimum(part, d[:, k * 128:(k + 1) * 128])
    d1_ref[0, 0] = jnp.maximum(jnp.min(part.T, axis=0), 0.0)
    colmin = jnp.min(d, axis=0)

    @pl.when(i == 0)
    def _():
        d2_ref[0, 0] = colmin

    @pl.when(i > 0)
    def _():
        d2_ref[0, 0] = jnp.minimum(d2_ref[0, 0], colmin)


def _tc_chamfer(p_t, g_t):
    return pl.pallas_call(
        _tc_body,
        grid=(_NB_TC, _NI),
        in_specs=[
            pl.BlockSpec((1, 3, _RT), lambda b, i: (b, 0, i)),
            pl.BlockSpec((1, 3, _M), lambda b, i: (b, 0, 0)),
        ],
        out_specs=[
            pl.BlockSpec((1, 1, _RT), lambda b, i: (b, 0, i)),
            pl.BlockSpec((1, 1, _M), lambda b, i: (b, 0, 0)),
        ],
        out_shape=[
            jax.ShapeDtypeStruct((_NB_TC, 1, _N), jnp.float32),
            jax.ShapeDtypeStruct((_NB_TC, 1, _M), jnp.float32),
        ],
    )(p_t, g_t)


def kernel(pred_points, gt_points):
    p_t = jnp.transpose(pred_points, (0, 2, 1))
    g_t = jnp.transpose(gt_points, (0, 2, 1))
    d1_sc, cm_sc = _sc_chamfer(p_t[:_NB_SC], g_t[:_NB_SC])
    d1_tc, d2_tc = _tc_chamfer(p_t[_NB_SC:], g_t[_NB_SC:])
    d2_sc = jnp.maximum(jnp.min(cm_sc, axis=1), 0.0)
    d1 = jnp.concatenate([d1_sc, d1_tc[:, 0, :]], axis=0)
    d2 = jnp.concatenate([d2_sc, jnp.maximum(d2_tc[:, 0, :], 0.0)], axis=0)
    return jnp.mean(d1) + jnp.mean(d2)
